# Initial kernel scaffold; baseline (speedup 1.0000x reference)
import jax
import jax.numpy as jnp
from jax.experimental import pallas as pl


def kernel(x, spiral, W, b):
    raise NotImplementedError("write your pallas kernel here")


# speedup vs baseline: 10.7486x; 10.7486x over previous
import functools

import jax
import jax.numpy as jnp
from jax import lax
from jax.experimental import pallas as pl
from jax.experimental.pallas import tpu as pltpu
from jax.experimental.pallas import tpu_sc as plsc

B = 8
V = 10000
S = 9
F = 128
OUT = 128

NC = 2
NS = 16
NW = NC * NS
CH = 128
NCHUNK = 80
VP = NCHUNK * CH
JOBS = S * B * NCHUNK
JOBS_PER_W = JOBS // NW


def _sc_gather(xflat, idxp):
    mesh = plsc.VectorSubcoreMesh(core_axis_name="c", subcore_axis_name="s")

    @functools.partial(
        pl.kernel,
        mesh=mesh,
        out_type=jax.ShapeDtypeStruct((S, B, VP, F), jnp.float32),
        scratch_types=[
            pltpu.VMEM((CH,), jnp.int32),
            pltpu.VMEM((CH,), jnp.int32),
            pltpu.VMEM((CH, F), jnp.float32),
            pltpu.SemaphoreType.DMA,
        ],
    )
    def k(xflat_hbm, idxp_hbm, g_hbm, idx_raw, idx_adj, rows, sem):
        cid = lax.axis_index("c")
        sid = lax.axis_index("s")
        wid = sid * NC + cid

        def job(t, carry):
            j = wid + NW * t
            s = j // (B * NCHUNK)
            r = j % (B * NCHUNK)
            bb = r // NCHUNK
            c = r % NCHUNK
            pltpu.sync_copy(idxp_hbm.at[s, pl.ds(c * CH, CH)], idx_raw)
            off = bb * (V + 1)
            for i in range(CH // 16):
                sl = pl.ds(i * 16, 16)
                idx_adj[sl] = idx_raw[sl] + off
            pltpu.async_copy(xflat_hbm.at[idx_adj], rows, sem).wait()
            pltpu.sync_copy(rows, g_hbm.at[s, bb, pl.ds(c * CH, CH), :])
            return carry

        lax.fori_loop(0, JOBS_PER_W, job, 0)

    return k(xflat, idxp)


def _tc_matmul(g, wt, bias):
    VB = 400

    def body(g_ref, w_ref, b_ref, o_ref):
        acc = jnp.dot(g_ref[0, 0], w_ref[0], preferred_element_type=jnp.float32)
        for s in range(1, S):
            acc += jnp.dot(g_ref[s, 0], w_ref[s], preferred_element_type=jnp.float32)
        o_ref[0] = jnp.maximum(acc + b_ref[0], 0.0)

    return pl.pallas_call(
        body,
        grid=(B, V // VB),
        in_specs=[
            pl.BlockSpec((S, 1, VB, F), lambda b, i: (0, b, i, 0)),
            pl.BlockSpec((S, F, OUT), lambda b, i: (0, 0, 0)),
            pl.BlockSpec((1, OUT), lambda b, i: (0, 0)),
        ],
        out_specs=pl.BlockSpec((1, VB, OUT), lambda b, i: (b, i, 0)),
        out_shape=jax.ShapeDtypeStruct((B, V, OUT), jnp.float32),
        compiler_params=pltpu.CompilerParams(
            dimension_semantics=("parallel", "parallel"),
        ),
    )(g, wt, bias)


@jax.jit
def kernel(x, spiral, W, b):
    xflat = jnp.concatenate(
        [x, jnp.zeros((B, 1, F), dtype=x.dtype)], axis=1
    ).reshape(B * (V + 1), F)
    idxp = jnp.pad(spiral[0, :V, :].T, ((0, 0), (0, VP - V)))
    g = _sc_gather(xflat, idxp)
    wt = W.reshape(OUT, S, F).transpose(1, 2, 0)
    return _tc_matmul(g, wt, b.reshape(1, OUT))


# speedup vs baseline: 16.2934x; 1.5159x over previous
import functools

import jax
import jax.numpy as jnp
from jax import lax
from jax.experimental import pallas as pl
from jax.experimental.pallas import tpu as pltpu
from jax.experimental.pallas import tpu_sc as plsc

B = 8
V = 10000
S = 9
F = 128
OUT = 128

NC = 2
NS = 16
NW = NC * NS
CH = 128
NCHUNK = 80
VP = NCHUNK * CH
JOBS = S * B * NCHUNK
JOBS_PER_W = JOBS // NW


NB = 4
KPW = 20


def _sc_gather(xflat, idxp4):
    mesh = plsc.VectorSubcoreMesh(core_axis_name="c", subcore_axis_name="s")

    @functools.partial(
        pl.kernel,
        mesh=mesh,
        out_type=jax.ShapeDtypeStruct((S, B, VP, F), jnp.float32),
        scratch_types=[
            pltpu.VMEM((KPW, CH), jnp.int32),
            [pltpu.VMEM((CH, F), jnp.float32) for _ in range(NB)],
            [pltpu.SemaphoreType.DMA for _ in range(NB)],
            [pltpu.SemaphoreType.DMA for _ in range(NB)],
        ],
    )
    def k(xflat_hbm, idxp_hbm, g_hbm, idxrow, rows, gsem, wsem):
        cid = lax.axis_index("c")
        sid = lax.axis_index("s")
        wid = sid * NC + cid
        bb = wid % B
        base_q = wid // B
        off = bb * (V + 1)

        def wb_wait(u):
            pltpu.make_async_copy(
                rows[u], g_hbm.at[0, 0, pl.ds(0, CH), :], wsem[u]
            ).wait()

        for s in range(S):
            pltpu.sync_copy(idxp_hbm.at[s, :, base_q, :], idxrow)

            def add_off(a, carry):
                for i in range(CH // 16):
                    sl = pl.ds(i * 16, 16)
                    idxrow[a, sl] = idxrow[a, sl] + off
                return carry

            lax.fori_loop(0, KPW, add_off, 0)

            def block(blk, carry):
                descs = []
                for u in range(NB):
                    if s == 0:
                        @pl.when(blk > 0)
                        def _():
                            wb_wait(u)
                    else:
                        wb_wait(u)
                    kk = blk * NB + u
                    descs.append(
                        pltpu.async_copy(xflat_hbm.at[idxrow.at[kk]], rows[u], gsem[u])
                    )
                for u in range(NB):
                    kk = blk * NB + u
                    c = base_q + 4 * kk
                    descs[u].wait()
                    pltpu.async_copy(
                        rows[u], g_hbm.at[s, bb, pl.ds(c * CH, CH), :], wsem[u]
                    )
                return carry

            lax.fori_loop(0, KPW // NB, block, 0)

        for u in range(NB):
            wb_wait(u)

    return k(xflat, idxp4)


def _tc_matmul(g, wt, bias):
    VB = 400

    def body(g_ref, w_ref, b_ref, o_ref):
        acc = jnp.dot(g_ref[0, 0], w_ref[0], preferred_element_type=jnp.float32)
        for s in range(1, S):
            acc += jnp.dot(g_ref[s, 0], w_ref[s], preferred_element_type=jnp.float32)
        o_ref[0] = jnp.maximum(acc + b_ref[0], 0.0)

    return pl.pallas_call(
        body,
        grid=(B, V // VB),
        in_specs=[
            pl.BlockSpec((S, 1, VB, F), lambda b, i: (0, b, i, 0)),
            pl.BlockSpec((S, F, OUT), lambda b, i: (0, 0, 0)),
            pl.BlockSpec((1, OUT), lambda b, i: (0, 0)),
        ],
        out_specs=pl.BlockSpec((1, VB, OUT), lambda b, i: (b, i, 0)),
        out_shape=jax.ShapeDtypeStruct((B, V, OUT), jnp.float32),
        compiler_params=pltpu.CompilerParams(
            dimension_semantics=("parallel", "parallel"),
        ),
    )(g, wt, bias)


@jax.jit
def kernel(x, spiral, W, b):
    xflat = jnp.concatenate(
        [x, jnp.zeros((B, 1, F), dtype=x.dtype)], axis=1
    ).reshape(B * (V + 1), F)
    idxp4 = jnp.pad(spiral[0, :V, :].T, ((0, 0), (0, VP - V))).reshape(
        S, KPW, 4, CH
    )
    g = _sc_gather(xflat, idxp4)
    wt = W.reshape(OUT, S, F).transpose(1, 2, 0)
    return _tc_matmul(g, wt, b.reshape(1, OUT))


# speedup vs baseline: 25.1105x; 1.5411x over previous
import functools

import jax
import jax.numpy as jnp
from jax import lax
from jax.experimental import pallas as pl
from jax.experimental.pallas import tpu as pltpu
from jax.experimental.pallas import tpu_sc as plsc

B = 8
V = 10000
S = 9
F = 128
OUT = 128

B2 = B // 2
NC = 2
NS = 16
NW = NC * NS
CH = 128
NCHUNK = 80
VP = NCHUNK * CH
NB = 5
KPW = 10


def _sc_gather(xpk, idxp):
    mesh = plsc.VectorSubcoreMesh(core_axis_name="c", subcore_axis_name="s")

    @functools.partial(
        pl.kernel,
        mesh=mesh,
        out_type=jax.ShapeDtypeStruct((S, B2, VP, F), jnp.int32),
        scratch_types=[
            pltpu.VMEM((KPW, CH), jnp.int32),
            [pltpu.VMEM((CH, F), jnp.int32) for _ in range(NB)],
            [pltpu.SemaphoreType.DMA for _ in range(NB)],
            [pltpu.SemaphoreType.DMA for _ in range(NB)],
        ],
    )
    def k(xpk_hbm, idxp_hbm, g_hbm, idxrow, rows, gsem, wsem):
        cid = lax.axis_index("c")
        sid = lax.axis_index("s")
        wid = sid * NC + cid
        pp = wid % B2
        base_q = wid // B2
        off = pp * (V + 1)

        def wb_wait(u):
            pltpu.make_async_copy(
                rows[u], g_hbm.at[0, 0, pl.ds(0, CH), :], wsem[u]
            ).wait()

        for s in range(S):
            pltpu.sync_copy(idxp_hbm.at[s, :, base_q, :], idxrow)

            def add_off(a, carry):
                for i in range(CH // 16):
                    sl = pl.ds(i * 16, 16)
                    idxrow[a, sl] = idxrow[a, sl] + off
                return carry

            lax.fori_loop(0, KPW, add_off, 0)

            def block(blk, carry):
                descs = []
                for u in range(NB):
                    if s == 0:
                        @pl.when(blk > 0)
                        def _():
                            wb_wait(u)
                    else:
                        wb_wait(u)
                    kk = blk * NB + u
                    descs.append(
                        pltpu.async_copy(xpk_hbm.at[idxrow.at[kk]], rows[u], gsem[u])
                    )
                for u in range(NB):
                    kk = blk * NB + u
                    c = base_q + 8 * kk
                    descs[u].wait()
                    pltpu.async_copy(
                        rows[u], g_hbm.at[s, pp, pl.ds(c * CH, CH), :], wsem[u]
                    )
                return carry

            lax.fori_loop(0, KPW // NB, block, 0)

        for u in range(NB):
            wb_wait(u)

    return k(xpk, idxp)


def _tc_matmul(g, wt, bias):
    VB = 400

    def body(g_ref, w_ref, b_ref, o_ref):
        acc0 = jnp.zeros((VB, OUT), jnp.float32)
        acc1 = jnp.zeros((VB, OUT), jnp.float32)
        for s in range(S):
            u = lax.bitcast_convert_type(g_ref[s, 0], jnp.uint32)
            lo = lax.bitcast_convert_type(u << 16, jnp.float32)
            hi = lax.bitcast_convert_type(u & jnp.uint32(0xFFFF0000), jnp.float32)
            acc0 += jnp.dot(
                lo.astype(jnp.bfloat16), w_ref[s], preferred_element_type=jnp.float32
            )
            acc1 += jnp.dot(
                hi.astype(jnp.bfloat16), w_ref[s], preferred_element_type=jnp.float32
            )
        o_ref[0] = jnp.maximum(acc0 + b_ref[0], 0.0)
        o_ref[1] = jnp.maximum(acc1 + b_ref[0], 0.0)

    return pl.pallas_call(
        body,
        grid=(B2, V // VB),
        in_specs=[
            pl.BlockSpec((S, 1, VB, F), lambda p, i: (0, p, i, 0)),
            pl.BlockSpec((S, F, OUT), lambda p, i: (0, 0, 0)),
            pl.BlockSpec((1, OUT), lambda p, i: (0, 0)),
        ],
        out_specs=pl.BlockSpec((2, VB, OUT), lambda p, i: (p, i, 0)),
        out_shape=jax.ShapeDtypeStruct((B, V, OUT), jnp.float32),
        compiler_params=pltpu.CompilerParams(
            dimension_semantics=("parallel", "parallel"),
        ),
    )(g, wt, bias)


@jax.jit
def kernel(x, spiral, W, b):
    xd = jnp.concatenate([x, jnp.zeros((B, 1, F), dtype=x.dtype)], axis=1)
    xb = lax.bitcast_convert_type(xd.astype(jnp.bfloat16), jnp.uint16).astype(
        jnp.uint32
    )
    xpk = lax.bitcast_convert_type(xb[0::2] | (xb[1::2] << 16), jnp.int32)
    xpk = xpk.reshape(B2 * (V + 1), F)
    idxp = jnp.pad(spiral[0, :V, :].T, ((0, 0), (0, VP - V))).reshape(
        S, KPW, 8, CH
    )
    g = _sc_gather(xpk, idxp)
    wt = W.reshape(OUT, S, F).transpose(1, 2, 0).astype(jnp.bfloat16)
    return _tc_matmul(g, wt, b.reshape(1, OUT))


# speedup vs baseline: 27.4948x; 1.0950x over previous
import functools

import jax
import jax.numpy as jnp
from jax import lax
from jax.experimental import pallas as pl
from jax.experimental.pallas import tpu as pltpu
from jax.experimental.pallas import tpu_sc as plsc

B = 8
V = 10000
S = 9
F = 128
OUT = 128

B2 = B // 2
NC = 2
NS = 16
NW = NC * NS
CH = 128
NCHUNK = 80
VP = NCHUNK * CH
NB = 6
KPW = 10
JOBW = S * KPW


def _sc_gather(xpk, idxp):
    mesh = plsc.VectorSubcoreMesh(core_axis_name="c", subcore_axis_name="s")

    @functools.partial(
        pl.kernel,
        mesh=mesh,
        out_type=jax.ShapeDtypeStruct((S, B2, VP, F), jnp.int32),
        scratch_types=[
            pltpu.VMEM((S, KPW, CH), jnp.int32),
            [pltpu.VMEM((CH, F), jnp.int32) for _ in range(NB)],
            [pltpu.SemaphoreType.DMA for _ in range(NB)],
            [pltpu.SemaphoreType.DMA for _ in range(NB)],
        ],
    )
    def k(xpk_hbm, idxp_hbm, g_hbm, idxall, rows, gsem, wsem):
        cid = lax.axis_index("c")
        sid = lax.axis_index("s")
        wid = sid * NC + cid
        pp = wid % B2
        base_q = wid // B2
        off = pp * (V + 1)

        pltpu.sync_copy(idxp_hbm.at[:, :, base_q, :], idxall)

        def add_off(a, carry):
            s = a // KPW
            kk = a % KPW
            for i in range(CH // 16):
                sl = pl.ds(i * 16, 16)
                idxall[s, kk, sl] = idxall[s, kk, sl] + off
            return carry

        lax.fori_loop(0, JOBW, add_off, 0)

        def wb_wait(u):
            pltpu.make_async_copy(
                rows[u], g_hbm.at[0, 0, pl.ds(0, CH), :], wsem[u]
            ).wait()

        def block(blk, carry):
            descs = []
            for u in range(NB):
                t = blk * NB + u
                s = t // KPW
                kk = t % KPW

                @pl.when(blk > 0)
                def _():
                    wb_wait(u)

                descs.append(
                    pltpu.async_copy(xpk_hbm.at[idxall.at[s, kk]], rows[u], gsem[u])
                )
            for u in range(NB):
                t = blk * NB + u
                s = t // KPW
                c = base_q + 8 * (t % KPW)
                descs[u].wait()
                pltpu.async_copy(
                    rows[u], g_hbm.at[s, pp, pl.ds(c * CH, CH), :], wsem[u]
                )
            return carry

        lax.fori_loop(0, JOBW // NB, block, 0)

        for u in range(NB):
            wb_wait(u)

    return k(xpk, idxp)


def _tc_matmul(g, wt, bias):
    VB = 1000

    def body(g_ref, w_ref, b_ref, o_ref):
        acc0 = jnp.zeros((VB, OUT), jnp.float32)
        acc1 = jnp.zeros((VB, OUT), jnp.float32)
        for s in range(S):
            u = lax.bitcast_convert_type(g_ref[s, 0], jnp.uint32)
            lo = lax.bitcast_convert_type(u << 16, jnp.float32)
            hi = lax.bitcast_convert_type(u & jnp.uint32(0xFFFF0000), jnp.float32)
            acc0 += jnp.dot(
                lo.astype(jnp.bfloat16), w_ref[s], preferred_element_type=jnp.float32
            )
            acc1 += jnp.dot(
                hi.astype(jnp.bfloat16), w_ref[s], preferred_element_type=jnp.float32
            )
        o_ref[0] = jnp.maximum(acc0 + b_ref[0], 0.0)
        o_ref[1] = jnp.maximum(acc1 + b_ref[0], 0.0)

    return pl.pallas_call(
        body,
        grid=(B2, V // VB),
        in_specs=[
            pl.BlockSpec((S, 1, VB, F), lambda p, i: (0, p, i, 0)),
            pl.BlockSpec((S, F, OUT), lambda p, i: (0, 0, 0)),
            pl.BlockSpec((1, OUT), lambda p, i: (0, 0)),
        ],
        out_specs=pl.BlockSpec((2, VB, OUT), lambda p, i: (p, i, 0)),
        out_shape=jax.ShapeDtypeStruct((B, V, OUT), jnp.float32),
        compiler_params=pltpu.CompilerParams(
            dimension_semantics=("parallel", "parallel"),
        ),
    )(g, wt, bias)


@jax.jit
def kernel(x, spiral, W, b):
    xb = lax.bitcast_convert_type(x.astype(jnp.bfloat16), jnp.uint16).astype(
        jnp.uint32
    )
    body = xb[0::2] | (xb[1::2] << 16)
    xpk = jnp.concatenate([body, jnp.zeros((B2, 1, F), jnp.uint32)], axis=1)
    xpk = lax.bitcast_convert_type(xpk, jnp.int32).reshape(B2 * (V + 1), F)
    idxp = jnp.pad(spiral[0, :V, :].T, ((0, 0), (0, VP - V))).reshape(
        S, KPW, 8, CH
    )
    g = _sc_gather(xpk, idxp)
    wt = W.reshape(OUT, S, F).transpose(1, 2, 0).astype(jnp.bfloat16)
    return _tc_matmul(g, wt, b.reshape(1, OUT))
